# baseline (speedup 1.0000x reference)
import jax
import jax.numpy as jnp
import numpy as np
from jax.experimental import pallas as pl
from jax.experimental.pallas import tpu as pltpu

C_IN, SEQ_LEN = 9, 2048
KERNEL_SIZE = 9
NUM_KERNELS = 84
NUM_FEATURES = 10000
MAX_DIL_PER_KERNEL = 32
BATCH = 32


def _cfg():
    nf_total = NUM_FEATURES // 2
    nf_total = nf_total // NUM_KERNELS * NUM_KERNELS
    nfpk = nf_total // NUM_KERNELS
    true_max = min(nfpk, MAX_DIL_PER_KERNEL)
    multiplier = nfpk / true_max
    max_exp = np.log2((SEQ_LEN - 1) / (KERNEL_SIZE - 1))
    dilations, counts = np.unique(
        np.logspace(0, max_exp, true_max, base=2).astype(np.int32),
        return_counts=True)
    nfpd = (counts * multiplier).astype(np.int32)
    rem = nfpk - nfpd.sum()
    i = 0
    while rem > 0:
        nfpd[i] += 1
        rem -= 1
        i = (i + 1) % len(nfpd)
    return [int(d) for d in dilations], [int(n) for n in nfpd]


_DILS, _NFPD = _cfg()
_D = len(_DILS)
_NF_SUM = sum(_NFPD)
_HALF = NUM_KERNELS // 2
_SLOTS = 2 * _NF_SUM
_OUT_W = 4 * _HALF * _NF_SUM

_PERM = list(range(0, NUM_KERNELS, 2)) + list(range(1, NUM_KERNELS, 2))


def _final_perm():
    pidx = np.zeros(_OUT_W, np.int32)
    base = 0
    boff = 0
    for i, nf in enumerate(_NFPD):
        p1 = i % 2
        c0 = 2 * boff
        rA = 0 if p1 == 0 else _HALF
        rB = _HALF - rA
        for seg, (r0, cadd) in enumerate(
                [(rA, 0), (rA, nf), (rB, 0), (rB, nf)]):
            for j in range(_HALF):
                for f in range(nf):
                    pidx[base + seg * _HALF * nf + j * nf + f] = (
                        (r0 + j) * _SLOTS + (c0 + cadd + f))
        base += 4 * _HALF * nf
        boff += nf
    return pidx


_PIDX = _final_perm()


_CHUNK = 128
_PACK = 8192.0


def _ppv_group(CK, bcols, nf, lo_pos, hi_pos):
    zero = jnp.zeros((_HALF, _CHUNK), jnp.float32)
    acc = [zero] * nf
    acc_t = zero
    bbK = [bcols[:, f:f + 1] + _PACK for f in range(nf)]
    j0 = lo_pos // _CHUNK
    j1 = (hi_pos + _CHUNK - 1) // _CHUNK
    for j in range(j0, j1):
        CKj = CK[:, j * _CHUNK:(j + 1) * _CHUNK]
        a = lo_pos - j * _CHUNK
        b = hi_pos - j * _CHUNK
        if a <= 0 and b >= _CHUNK:
            vmask = None
            acc_t = acc_t + CKj
        else:
            io = jax.lax.broadcasted_iota(jnp.int32, (1, _CHUNK), 1)
            vmask = (io >= a) & (io < b)
            acc_t = acc_t + jnp.where(vmask, CKj, 0.0)
        for f in range(nf):
            m = CKj > bbK[f]
            if vmask is not None:
                m = m & vmask
            acc[f] = acc[f] + jnp.where(m, CKj, 0.0)
    cnt = []
    S = []
    for f in range(nf):
        nl = jnp.round(acc[f] * (1.0 / _PACK))
        Sl = acc[f] - nl * _PACK
        cnt.append(jnp.sum(nl, axis=1, keepdims=True))
        S.append(jnp.sum(Sl, axis=1, keepdims=True))
    ntl = jnp.round(acc_t * (1.0 / _PACK))
    tot = jnp.sum(acc_t - ntl * _PACK, axis=1, keepdims=True)
    return cnt, S, tot


_ROWS = 16


_BB = 4


def _body(x_ref, w_ref, b_ref, o_ref):
    L = SEQ_LEN
    xb = [jnp.concatenate(
        [x_ref[bi], jnp.ones((1, SEQ_LEN), jnp.float32),
         jnp.zeros((_ROWS - C_IN - 1, SEQ_LEN), jnp.float32)],
        axis=0) for bi in range(_BB)]
    boff = 0
    for i, (d, nf) in enumerate(zip(_DILS, _NFPD)):
        p = (KERNEL_SIZE - 1) * d // 2
        p1 = i % 2
        gF = p1
        gC = 1 - p1
        Lc = L - 2 * p
        c0 = 2 * boff
        bF = b_ref[gF, :, boff:boff + nf]
        bC = b_ref[gC, :, boff:boff + nf]
        for bi in range(_BB):
            x = xb[bi]
            shifts = []
            for t in range(KERNEL_SIZE):
                o = (t - KERNEL_SIZE // 2) * d
                if o == 0:
                    shifts.append(x)
                elif o > 0:
                    z = jnp.zeros((_ROWS, o), jnp.float32)
                    shifts.append(jnp.concatenate([x[:, o:], z], axis=1))
                else:
                    z = jnp.zeros((_ROWS, -o), jnp.float32)
                    shifts.append(jnp.concatenate([z, x[:, :o]], axis=1))
            xs = jnp.concatenate(shifts, axis=0)

            Cf = jnp.dot(w_ref[i, gF], xs,
                         preferred_element_type=jnp.float32)
            Cc = jnp.dot(w_ref[i, gC], xs,
                         preferred_element_type=jnp.float32)

            cntF, SF, sf = _ppv_group(Cf, bF, nf, 0, L)
            cntC, SC, sc = _ppv_group(Cc, bC, nf, p, L - p)

            colsF = []
            colsC = []
            bfF_cols = []
            bfC_cols = []
            for f in range(nf):
                b1 = bF[:, f:f + 1]
                reluF = SF[f] - cntF[f] * b1
                absF = 2.0 * reluF - (sf - L * b1)
                colsF.append(cntF[f] * (1.0 / L))
                bfF_cols.append(reluF / jnp.maximum(absF, 1e-8))

                b2 = bC[:, f:f + 1]
                reluC = SC[f] - cntC[f] * b2
                absC = 2.0 * reluC - (sc - Lc * b2)
                colsC.append(cntC[f] * (1.0 / Lc))
                bfC_cols.append(reluC / jnp.maximum(absC, 1e-8))
            o_ref[bi, gF, :, c0:c0 + 2 * nf] = jnp.concatenate(
                colsF + bfF_cols, axis=1)
            o_ref[bi, gC, :, c0:c0 + 2 * nf] = jnp.concatenate(
                colsC + bfC_cols, axis=1)
        boff += nf


def kernel(x, kernels, channel_combinations, biases):
    B = x.shape[0]
    kw = kernels[:NUM_KERNELS, 0, :]
    cc = channel_combinations.transpose(0, 2, 1)
    W = kw[None, :, :, None] * cc[:, :, None, :]
    W = jnp.pad(W, ((0, 0), (0, 0), (0, 0), (0, _ROWS - C_IN)))
    W = W.at[:, :, KERNEL_SIZE // 2, C_IN].set(_PACK)
    W = W.reshape(_D, NUM_KERNELS, KERNEL_SIZE * _ROWS)
    W = W[:, jnp.asarray(_PERM, jnp.int32), :]
    W = W.reshape(_D, 2, _HALF, KERNEL_SIZE * _ROWS)
    b_perm = biases[jnp.asarray(_PERM, jnp.int32), :]
    b_perm = b_perm.reshape(2, _HALF, _NF_SUM)

    staged = pl.pallas_call(
        _body,
        out_shape=jax.ShapeDtypeStruct((B, 2, _HALF, _SLOTS), jnp.float32),
        grid=(B // _BB,),
        in_specs=[
            pl.BlockSpec((_BB, C_IN, SEQ_LEN), lambda i: (i, 0, 0)),
            pl.BlockSpec((_D, 2, _HALF, KERNEL_SIZE * _ROWS),
                         lambda i: (0, 0, 0, 0)),
            pl.BlockSpec((2, _HALF, _NF_SUM), lambda i: (0, 0, 0)),
        ],
        out_specs=pl.BlockSpec((_BB, 2, _HALF, _SLOTS),
                               lambda i: (i, 0, 0, 0)),
        compiler_params=pltpu.CompilerParams(
            dimension_semantics=("parallel",),
        ),
        name="minirocket_features",
    )(x, W, b_perm)

    return staged.reshape(B, NUM_KERNELS * _SLOTS)[:, jnp.asarray(_PIDX)]


# speedup vs baseline: 1.2002x; 1.2002x over previous
import jax
import jax.numpy as jnp
import numpy as np
from jax.experimental import pallas as pl
from jax.experimental.pallas import tpu as pltpu

C_IN, SEQ_LEN = 9, 2048
KERNEL_SIZE = 9
NUM_KERNELS = 84
NUM_FEATURES = 10000
MAX_DIL_PER_KERNEL = 32
BATCH = 32


def _cfg():
    nf_total = NUM_FEATURES // 2
    nf_total = nf_total // NUM_KERNELS * NUM_KERNELS
    nfpk = nf_total // NUM_KERNELS
    true_max = min(nfpk, MAX_DIL_PER_KERNEL)
    multiplier = nfpk / true_max
    max_exp = np.log2((SEQ_LEN - 1) / (KERNEL_SIZE - 1))
    dilations, counts = np.unique(
        np.logspace(0, max_exp, true_max, base=2).astype(np.int32),
        return_counts=True)
    nfpd = (counts * multiplier).astype(np.int32)
    rem = nfpk - nfpd.sum()
    i = 0
    while rem > 0:
        nfpd[i] += 1
        rem -= 1
        i = (i + 1) % len(nfpd)
    return [int(d) for d in dilations], [int(n) for n in nfpd]


_DILS, _NFPD = _cfg()
_D = len(_DILS)
_NF_SUM = sum(_NFPD)
_HALF = NUM_KERNELS // 2
_SLOTS = 2 * _NF_SUM
_OUT_W = 4 * _HALF * _NF_SUM

_PERM = list(range(0, NUM_KERNELS, 2)) + list(range(1, NUM_KERNELS, 2))


def _final_perm():
    pidx = np.zeros(_OUT_W, np.int32)
    base = 0
    boff = 0
    for i, nf in enumerate(_NFPD):
        p1 = i % 2
        c0 = 2 * boff
        rA = 0 if p1 == 0 else _HALF
        rB = _HALF - rA
        for seg, (r0, cadd) in enumerate(
                [(rA, 0), (rA, nf), (rB, 0), (rB, nf)]):
            for j in range(_HALF):
                for f in range(nf):
                    pidx[base + seg * _HALF * nf + j * nf + f] = (
                        (r0 + j) * _SLOTS + (c0 + cadd + f))
        base += 4 * _HALF * nf
        boff += nf
    return pidx


_PIDX = _final_perm()


_CHUNK = 128
_PACK = 8192.0


def _ppv_group(CK, bcols, nf, lo_pos, hi_pos):
    zero = jnp.zeros((_HALF, _CHUNK), jnp.float32)
    acc = [zero] * nf
    acc_t = zero
    bbK = [bcols[:, f:f + 1] + _PACK for f in range(nf)]
    j0 = lo_pos // _CHUNK
    j1 = (hi_pos + _CHUNK - 1) // _CHUNK
    for j in range(j0, j1):
        CKj = CK[:, j * _CHUNK:(j + 1) * _CHUNK]
        a = lo_pos - j * _CHUNK
        b = hi_pos - j * _CHUNK
        if a <= 0 and b >= _CHUNK:
            vmask = None
            acc_t = acc_t + CKj
        else:
            io = jax.lax.broadcasted_iota(jnp.int32, (1, _CHUNK), 1)
            vmask = (io >= a) & (io < b)
            acc_t = acc_t + jnp.where(vmask, CKj, 0.0)
        for f in range(nf):
            m = CKj > bbK[f]
            if vmask is not None:
                m = m & vmask
            acc[f] = acc[f] + jnp.where(m, CKj, 0.0)
    cnt = []
    S = []
    for f in range(nf):
        nl = jnp.round(acc[f] * (1.0 / _PACK))
        Sl = acc[f] - nl * _PACK
        cnt.append(jnp.sum(nl, axis=1, keepdims=True))
        S.append(jnp.sum(Sl, axis=1, keepdims=True))
    ntl = jnp.round(acc_t * (1.0 / _PACK))
    tot = jnp.sum(acc_t - ntl * _PACK, axis=1, keepdims=True)
    return cnt, S, tot


_ROWS = 16


_BB = 2


def _body(x_ref, w_ref, b_ref, o_ref):
    L = SEQ_LEN
    xb = [jnp.concatenate(
        [x_ref[bi].astype(jnp.bfloat16),
         jnp.ones((1, SEQ_LEN), jnp.bfloat16),
         jnp.zeros((_ROWS - C_IN - 1, SEQ_LEN), jnp.bfloat16)],
        axis=0) for bi in range(_BB)]
    boff = 0
    for i, (d, nf) in enumerate(zip(_DILS, _NFPD)):
        p = (KERNEL_SIZE - 1) * d // 2
        p1 = i % 2
        gF = p1
        gC = 1 - p1
        Lc = L - 2 * p
        c0 = 2 * boff
        bF = b_ref[gF, :, boff:boff + nf]
        bC = b_ref[gC, :, boff:boff + nf]
        for bi in range(_BB):
            x = xb[bi]
            shifts = []
            for t in range(KERNEL_SIZE):
                o = (t - KERNEL_SIZE // 2) * d
                if o == 0:
                    shifts.append(x)
                elif o > 0:
                    z = jnp.zeros((_ROWS, o), jnp.bfloat16)
                    shifts.append(jnp.concatenate([x[:, o:], z], axis=1))
                else:
                    z = jnp.zeros((_ROWS, -o), jnp.bfloat16)
                    shifts.append(jnp.concatenate([z, x[:, :o]], axis=1))
            xs = jnp.concatenate(shifts, axis=0)

            Cf = jnp.dot(w_ref[i, gF], xs,
                         preferred_element_type=jnp.float32)
            Cc = jnp.dot(w_ref[i, gC], xs,
                         preferred_element_type=jnp.float32)

            cntF, SF, sf = _ppv_group(Cf, bF, nf, 0, L)
            cntC, SC, sc = _ppv_group(Cc, bC, nf, p, L - p)

            colsF = []
            colsC = []
            bfF_cols = []
            bfC_cols = []
            for f in range(nf):
                b1 = bF[:, f:f + 1]
                reluF = SF[f] - cntF[f] * b1
                absF = 2.0 * reluF - (sf - L * b1)
                colsF.append(cntF[f] * (1.0 / L))
                bfF_cols.append(reluF / jnp.maximum(absF, 1e-8))

                b2 = bC[:, f:f + 1]
                reluC = SC[f] - cntC[f] * b2
                absC = 2.0 * reluC - (sc - Lc * b2)
                colsC.append(cntC[f] * (1.0 / Lc))
                bfC_cols.append(reluC / jnp.maximum(absC, 1e-8))
            o_ref[bi, gF, :, c0:c0 + 2 * nf] = jnp.concatenate(
                colsF + bfF_cols, axis=1)
            o_ref[bi, gC, :, c0:c0 + 2 * nf] = jnp.concatenate(
                colsC + bfC_cols, axis=1)
        boff += nf


def kernel(x, kernels, channel_combinations, biases):
    B = x.shape[0]
    kw = kernels[:NUM_KERNELS, 0, :]
    cc = channel_combinations.transpose(0, 2, 1)
    W = kw[None, :, :, None] * cc[:, :, None, :]
    W = jnp.pad(W, ((0, 0), (0, 0), (0, 0), (0, _ROWS - C_IN)))
    W = W.at[:, :, KERNEL_SIZE // 2, C_IN].set(_PACK)
    W = W.reshape(_D, NUM_KERNELS, KERNEL_SIZE * _ROWS)
    W = W[:, jnp.asarray(_PERM, jnp.int32), :]
    W = W.reshape(_D, 2, _HALF, KERNEL_SIZE * _ROWS)
    W = W.astype(jnp.bfloat16)
    b_perm = biases[jnp.asarray(_PERM, jnp.int32), :]
    b_perm = b_perm.reshape(2, _HALF, _NF_SUM)

    staged = pl.pallas_call(
        _body,
        out_shape=jax.ShapeDtypeStruct((B, 2, _HALF, _SLOTS), jnp.float32),
        grid=(B // _BB,),
        in_specs=[
            pl.BlockSpec((_BB, C_IN, SEQ_LEN), lambda i: (i, 0, 0)),
            pl.BlockSpec((_D, 2, _HALF, KERNEL_SIZE * _ROWS),
                         lambda i: (0, 0, 0, 0)),
            pl.BlockSpec((2, _HALF, _NF_SUM), lambda i: (0, 0, 0)),
        ],
        out_specs=pl.BlockSpec((_BB, 2, _HALF, _SLOTS),
                               lambda i: (i, 0, 0, 0)),
        compiler_params=pltpu.CompilerParams(
            dimension_semantics=("parallel",),
        ),
        name="minirocket_features",
    )(x, W, b_perm)

    return staged.reshape(B, NUM_KERNELS * _SLOTS)[:, jnp.asarray(_PIDX)]
